# baseline (speedup 1.0000x reference)
import functools

import jax
import jax.numpy as jnp
from jax import lax
from jax.experimental import pallas as pl
from jax.experimental.pallas import tpu as pltpu
from jax.experimental.pallas import tpu_sc as plsc

CTX_DIM = 512
N_CLS_CTX = 4
N_PRE = 5
TOK_LEN = 77
N_SUF = TOK_LEN - N_PRE - N_CLS_CTX
LANES = 16
CHUNK = 8
SPLIT = 16
NBUF = 4


def _sc_head(table3d, label, template, b):
    info = plsc.get_sparse_core_info()
    num_workers = info.num_cores * info.num_subcores
    assert b % num_workers == 0
    bpw = b // num_workers
    assert bpw % CHUNK == 0 and CHUNK % NBUF == 0
    n_chunks = bpw // CHUNK
    lane_steps = CTX_DIM // LANES

    mesh = plsc.VectorSubcoreMesh(core_axis_name="c", subcore_axis_name="s")

    @functools.partial(
        pl.kernel,
        mesh=mesh,
        out_type=jax.ShapeDtypeStruct((b, TOK_LEN, CTX_DIM), jnp.float32),
        scratch_types=[
            pltpu.VMEM((bpw,), jnp.int32),
            pltpu.VMEM((CHUNK, N_CLS_CTX, CTX_DIM), jnp.float32),
            pltpu.VMEM((CHUNK, N_CLS_CTX, CTX_DIM), jnp.float32),
        ] + [pltpu.VMEM((SPLIT, CTX_DIM), jnp.float32)] * NBUF + [
            pltpu.SemaphoreType.DMA,
            pltpu.SemaphoreType.DMA,
        ] + [pltpu.SemaphoreType.DMA] * NBUF + [
            pltpu.SemaphoreType.DMA,
        ],
    )
    def body(table_hbm, idx_hbm, tmpl_hbm, out_hbm,
             idx_v, rga, rgb, blk0, blk1, blk2, blk3,
             gsema, gsemb, osem0, osem1, osem2, osem3, tsem):
        blks = (blk0, blk1, blk2, blk3)
        osems = (osem0, osem1, osem2, osem3)
        wid = lax.axis_index("s") * info.num_cores + lax.axis_index("c")
        base = wid * bpw
        pltpu.sync_copy(idx_hbm.at[pl.ds(base, bpw)], idx_v)
        tmpl_cps = [
            pltpu.make_async_copy(tmpl_hbm.at[pl.ds(0, SPLIT)], blk, tsem)
            for blk in blks
        ]
        for cp in tmpl_cps:
            cp.start()
        for cp in tmpl_cps:
            cp.wait()

        gather_bufs = (rga, rgb)
        gather_sems = (gsema, gsemb)

        def start_gather(c):
            pltpu.make_async_copy(
                table_hbm.at[idx_v.at[pl.ds(c * CHUNK, CHUNK)]],
                gather_bufs[c % 2], gather_sems[c % 2]).start()

        start_gather(0)
        for c in range(n_chunks):
            rg = gather_bufs[c % 2]
            pltpu.make_async_copy(
                table_hbm.at[idx_v.at[pl.ds(c * CHUNK, CHUNK)]],
                rg, gather_sems[c % 2]).wait()
            if c + 1 < n_chunks:
                start_gather(c + 1)

            def do_quad(t, _):
                for i in range(NBUF):
                    k = NBUF * t + i
                    j = c * CHUNK + k

                    @pl.when(j >= NBUF)
                    def _wait_prev():
                        pltpu.make_async_copy(
                            blks[i], out_hbm.at[base, pl.ds(0, SPLIT)],
                            osems[i]).wait()

                    for r in range(N_CLS_CTX):
                        for cc in range(lane_steps):
                            sl = pl.ds(cc * LANES, LANES)
                            blks[i][N_PRE + r, sl] = rg[k, r, sl]
                    pltpu.make_async_copy(
                        blks[i], out_hbm.at[base + j, pl.ds(0, SPLIT)],
                        osems[i]).start()
                return _

            lax.fori_loop(0, CHUNK // NBUF, do_quad, 0)

        for i in range(NBUF):
            pltpu.make_async_copy(
                blks[i], out_hbm.at[base, pl.ds(0, SPLIT)], osems[i]).wait()

    return body(table3d, label, template)


def _tc_tail(buf, token_suffix, group=64):
    b = buf.shape[0]
    assert b % group == 0
    n_groups = b // group
    tail_rows = TOK_LEN - SPLIT
    suf_off = SPLIT - N_PRE - N_CLS_CTX

    def body(buf_ref, suf_ref, out_ref, suf_v, sem):
        del buf_ref
        suf_v[...] = jnp.broadcast_to(
            suf_ref[:, suf_off:N_SUF, :], (group, tail_rows, CTX_DIM))
        copies = [
            pltpu.make_async_copy(
                suf_v,
                out_ref.at[pl.ds(i * group, group), pl.ds(SPLIT, tail_rows)],
                sem)
            for i in range(n_groups)
        ]
        for cp in copies:
            cp.start()
        for cp in copies:
            cp.wait()

    return pl.pallas_call(
        body,
        in_specs=[
            pl.BlockSpec(memory_space=pl.ANY),
            pl.BlockSpec((1, N_SUF, CTX_DIM), lambda: (0, 0, 0)),
        ],
        out_specs=pl.BlockSpec(memory_space=pl.ANY),
        out_shape=jax.ShapeDtypeStruct((b, TOK_LEN, CTX_DIM), jnp.float32),
        scratch_shapes=[
            pltpu.VMEM((group, tail_rows, CTX_DIM), jnp.float32),
            pltpu.SemaphoreType.DMA,
        ],
        input_output_aliases={0: 0},
    )(buf, token_suffix)


def kernel(label, cls_ctx, token_prefix, token_suffix):
    b = label.shape[0]
    template = jnp.zeros((SPLIT, CTX_DIM), jnp.float32)
    template = template.at[0:N_PRE].set(token_prefix[0])
    template = template.at[N_PRE + N_CLS_CTX:SPLIT].set(
        token_suffix[0, :SPLIT - N_PRE - N_CLS_CTX])
    buf = _sc_head(cls_ctx, label.astype(jnp.int32), template, b)
    return _tc_tail(buf, token_suffix)
